# baseline (speedup 1.0000x reference)
import functools

import jax
import jax.numpy as jnp
from jax import lax
from jax.experimental import pallas as pl
from jax.experimental.pallas import tpu as pltpu
from jax.experimental.pallas import tpu_sc as plsc

N_NODES = 10000
N_EDGES = 320000
DIM = 128
N_LAYERS = 3
N_GRAPHS = 64

NC = 2
NS = 16
NW = NC * NS
EDGES_PER_TILE = N_EDGES // NW
EDGE_BLK = 128
EPT_PAD = 10112
N_IT = EPT_PAD // EDGE_BLK
PAD_PACKED = 10239
N_PAD = 10240
ROWS_PER_TILE = N_PAD // NS
ZERO_ROWS = 16

ROW_BLK = 1000
GRID = N_NODES // ROW_BLK


def _sc_aggregate(cur, packed3):
    mesh = plsc.VectorSubcoreMesh(core_axis_name="c", subcore_axis_name="s")

    @functools.partial(
        pl.kernel,
        out_type=jax.ShapeDtypeStruct((NC, N_PAD, DIM), jnp.float32),
        mesh=mesh,
        scratch_types=[
            pltpu.VMEM_SHARED((N_PAD, DIM), jnp.float32),
            pltpu.VMEM((EDGE_BLK,), jnp.int32),
            pltpu.VMEM((EDGE_BLK,), jnp.int32),
            pltpu.VMEM((EDGE_BLK,), jnp.int32),
            pltpu.VMEM((EDGE_BLK,), jnp.int32),
            pltpu.VMEM((EDGE_BLK,), jnp.int32),
            pltpu.VMEM((EDGE_BLK,), jnp.int32),
            pltpu.VMEM((EDGE_BLK, DIM), jnp.float32),
            pltpu.VMEM((EDGE_BLK, DIM), jnp.float32),
            pltpu.SemaphoreType.DMA,
            pltpu.SemaphoreType.DMA,
            pltpu.SemaphoreType.DMA,
            pltpu.SemaphoreType.DMA,
        ],
    )
    def agg_kernel(cur_hbm, packed_hbm, out_hbm,
                   agg_sh, pk0, pk1, sb0, db0, sb1, db1, rows0, rows1,
                   isem0, isem1, sem0, sem1):
        c = lax.axis_index("c")
        s = lax.axis_index("s")
        wid = s * NC + c

        @pl.loop(0, EDGE_BLK)
        def _(r):
            @pl.loop(0, DIM, step=16)
            def _(col):
                rows0[r, pl.ds(col, 16)] = jnp.zeros((16,), jnp.float32)

        @pl.loop(0, ROWS_PER_TILE // EDGE_BLK)
        def _(j):
            pltpu.sync_copy(
                rows0,
                agg_sh.at[pl.ds(s * ROWS_PER_TILE + j * EDGE_BLK, EDGE_BLK)])

        plsc.subcore_barrier()

        def start_idx(j, pk, isem):
            pltpu.async_copy(packed_hbm.at[wid, j], pk, isem)

        def wait_idx(pk, isem):
            pltpu.make_async_copy(packed_hbm.at[0, 0], pk, isem).wait()

        def unpack(pk, sb, db):
            @pl.loop(0, EDGE_BLK, step=16)
            def _(j):
                v = pk[pl.ds(j, 16)]
                sb[pl.ds(j, 16)] = jnp.right_shift(v, 16)
                db[pl.ds(j, 16)] = jnp.bitwise_and(v, 0xFFFF)

        def start_gather(sb, rows, sem):
            pltpu.async_copy(cur_hbm.at[sb], rows, sem)

        def wait_gather(rows, sem):
            pltpu.make_async_copy(
                cur_hbm.at[pl.ds(0, EDGE_BLK)], rows, sem).wait()

        def scatter(rows, db):
            pltpu.sync_copy(rows, agg_sh.at[db], add=True)

        start_idx(0, pk0, isem0)
        start_idx(1, pk1, isem1)
        wait_idx(pk0, isem0)
        unpack(pk0, sb0, db0)
        start_idx(2, pk0, isem0)
        start_gather(sb0, rows0, sem0)

        @pl.loop(0, (N_IT - 3) // 2)
        def _(g):
            i = 2 * g
            wait_idx(pk1, isem1)
            unpack(pk1, sb1, db1)
            start_idx(i + 3, pk1, isem1)
            start_gather(sb1, rows1, sem1)
            wait_gather(rows0, sem0)
            scatter(rows0, db0)
            wait_idx(pk0, isem0)
            unpack(pk0, sb0, db0)
            start_idx(i + 4, pk0, isem0)
            start_gather(sb0, rows0, sem0)
            wait_gather(rows1, sem1)
            scatter(rows1, db1)

        wait_idx(pk1, isem1)
        unpack(pk1, sb1, db1)
        start_gather(sb1, rows1, sem1)
        wait_gather(rows0, sem0)
        scatter(rows0, db0)
        wait_idx(pk0, isem0)
        unpack(pk0, sb0, db0)
        start_gather(sb0, rows0, sem0)
        wait_gather(rows1, sem1)
        scatter(rows1, db1)
        wait_gather(rows0, sem0)
        scatter(rows0, db0)

        plsc.subcore_barrier()

        pltpu.sync_copy(
            agg_sh.at[pl.ds(s * ROWS_PER_TILE, ROWS_PER_TILE)],
            out_hbm.at[c, pl.ds(s * ROWS_PER_TILE, ROWS_PER_TILE)])

    return agg_kernel(cur, packed3)


def _mlp_layer(cur, parts, w1, b1, w2, b2, scale, relu):
    def body(scale_ref, cur_ref, p_ref, w1_ref, b1_ref, w2_ref, b2_ref,
             out_ref):
        sc = scale_ref[0]
        p = p_ref[...]
        h = cur_ref[...] * sc + p[0] + p[1]
        h = jnp.dot(h, w1_ref[...], preferred_element_type=jnp.float32)
        h = jnp.maximum(h + b1_ref[...], 0.0)
        h = jnp.dot(h, w2_ref[...], preferred_element_type=jnp.float32)
        h = h + b2_ref[...]
        if relu:
            h = jnp.maximum(h, 0.0)
        out_ref[...] = h

    return pl.pallas_call(
        body,
        grid=(GRID,),
        in_specs=[
            pl.BlockSpec(memory_space=pltpu.SMEM),
            pl.BlockSpec((ROW_BLK, DIM), lambda i: (i, 0)),
            pl.BlockSpec((NC, ROW_BLK, DIM), lambda i: (0, i, 0)),
            pl.BlockSpec((DIM, DIM), lambda i: (0, 0)),
            pl.BlockSpec((1, DIM), lambda i: (0, 0)),
            pl.BlockSpec((DIM, DIM), lambda i: (0, 0)),
            pl.BlockSpec((1, DIM), lambda i: (0, 0)),
        ],
        out_specs=pl.BlockSpec((ROW_BLK, DIM), lambda i: (i, 0)),
        out_shape=jax.ShapeDtypeStruct((N_NODES, DIM), jnp.float32),
    )(scale, cur, parts, w1, b1, w2, b2)


def _mlp_final_pool(cur, parts, w1, b1, w2, b2, scale, z1, z2, batch3):
    def body(scale_ref, cur_ref, p_ref, w1_ref, b1_ref, w2_ref, b2_ref,
             b_ref, z1_ref, z2_ref, z_out, g_out, g_acc, c_acc):
        k = pl.program_id(0)
        sc = scale_ref[0]
        p = p_ref[...]
        h = cur_ref[...] * sc + p[0] + p[1]
        h = jnp.dot(h, w1_ref[...], preferred_element_type=jnp.float32)
        h = jnp.maximum(h + b1_ref[...], 0.0)
        h = jnp.dot(h, w2_ref[...], preferred_element_type=jnp.float32)
        h = h + b2_ref[...]
        zb = (z1_ref[...] + z2_ref[...] + h) * (1.0 / 3.0)
        z_out[...] = zb
        bb = b_ref[0]
        oh = (lax.broadcasted_iota(jnp.int32, (N_GRAPHS, ROW_BLK), 0)
              == jnp.broadcast_to(bb, (N_GRAPHS, ROW_BLK))).astype(jnp.float32)

        @pl.when(k == 0)
        def _():
            g_acc[...] = jnp.zeros_like(g_acc)
            c_acc[...] = jnp.zeros_like(c_acc)

        g_acc[...] += jnp.dot(oh, zb, preferred_element_type=jnp.float32)
        c_acc[...] += jnp.broadcast_to(
            jnp.sum(oh, axis=1, keepdims=True), (N_GRAPHS, DIM))
        g_out[...] = g_acc[...] / jnp.maximum(c_acc[...], 1.0)

    return pl.pallas_call(
        body,
        grid=(GRID,),
        in_specs=[
            pl.BlockSpec(memory_space=pltpu.SMEM),
            pl.BlockSpec((ROW_BLK, DIM), lambda i: (i, 0)),
            pl.BlockSpec((NC, ROW_BLK, DIM), lambda i: (0, i, 0)),
            pl.BlockSpec((DIM, DIM), lambda i: (0, 0)),
            pl.BlockSpec((1, DIM), lambda i: (0, 0)),
            pl.BlockSpec((DIM, DIM), lambda i: (0, 0)),
            pl.BlockSpec((1, DIM), lambda i: (0, 0)),
            pl.BlockSpec((1, 1, ROW_BLK), lambda i: (i, 0, 0)),
            pl.BlockSpec((ROW_BLK, DIM), lambda i: (i, 0)),
            pl.BlockSpec((ROW_BLK, DIM), lambda i: (i, 0)),
        ],
        out_specs=[
            pl.BlockSpec((ROW_BLK, DIM), lambda i: (i, 0)),
            pl.BlockSpec((N_GRAPHS, DIM), lambda i: (0, 0)),
        ],
        out_shape=[
            jax.ShapeDtypeStruct((N_NODES, DIM), jnp.float32),
            jax.ShapeDtypeStruct((N_GRAPHS, DIM), jnp.float32),
        ],
        scratch_shapes=[
            pltpu.VMEM((N_GRAPHS, DIM), jnp.float32),
            pltpu.VMEM((N_GRAPHS, DIM), jnp.float32),
        ],
    )(scale, cur, parts, w1, b1, w2, b2, batch3, z1, z2)


def kernel(x, edge_index, batch, W1, b1, W2, b2, eps):
    packed = ((edge_index[0] << 16) | edge_index[1]).reshape(
        NW, EDGES_PER_TILE)
    pad = jnp.full((NW, EPT_PAD - EDGES_PER_TILE), PAD_PACKED, jnp.int32)
    packed3 = jnp.concatenate([packed, pad], axis=1).reshape(
        NW, N_IT, EDGE_BLK)
    batch3 = batch.reshape(GRID, 1, ROW_BLK)
    scales = (1.0 + eps).astype(jnp.float32)

    cur = x
    outs = []
    for i in range(N_LAYERS - 1):
        parts = _sc_aggregate(cur, packed3)
        cur = _mlp_layer(cur, parts, W1[i], b1[i].reshape(1, DIM),
                         W2[i], b2[i].reshape(1, DIM),
                         scales[i].reshape(1), relu=True)
        outs.append(cur)

    i = N_LAYERS - 1
    parts = _sc_aggregate(cur, packed3)
    z, g = _mlp_final_pool(cur, parts, W1[i], b1[i].reshape(1, DIM),
                           W2[i], b2[i].reshape(1, DIM),
                           scales[i].reshape(1), outs[0], outs[1], batch3)
    return (z, g)


# speedup vs baseline: 1.8364x; 1.8364x over previous
import functools

import jax
import jax.numpy as jnp
from jax import lax
from jax.experimental import pallas as pl
from jax.experimental.pallas import tpu as pltpu
from jax.experimental.pallas import tpu_sc as plsc

N_NODES = 10000
N_EDGES = 320000
DIM = 128
N_LAYERS = 3
N_GRAPHS = 64

NC = 2
NS = 16
NW = NC * NS
EDGES_PER_TILE = N_EDGES // NW
EDGE_BLK = 80
N_IT = EDGES_PER_TILE // EDGE_BLK
N_PAD = 10240
ROWS_PER_TILE = N_PAD // NS
ZERO_ROWS = 16

ROW_BLK = 1000
GRID = N_NODES // ROW_BLK


def _sc_aggregate(cur, packed3):
    mesh = plsc.VectorSubcoreMesh(core_axis_name="c", subcore_axis_name="s")

    @functools.partial(
        pl.kernel,
        out_type=jax.ShapeDtypeStruct((NC, N_PAD, DIM), jnp.float32),
        mesh=mesh,
        scratch_types=[
            pltpu.VMEM_SHARED((N_PAD, DIM), jnp.float32),
            pltpu.VMEM((N_IT, EDGE_BLK), jnp.int32),
            pltpu.VMEM((EDGE_BLK,), jnp.int32),
            pltpu.VMEM((EDGE_BLK,), jnp.int32),
            pltpu.VMEM((EDGE_BLK,), jnp.int32),
            pltpu.VMEM((EDGE_BLK,), jnp.int32),
            pltpu.VMEM((EDGE_BLK, DIM), jnp.float32),
            pltpu.VMEM((EDGE_BLK, DIM), jnp.float32),
            pltpu.SemaphoreType.DMA,
            pltpu.SemaphoreType.DMA,
        ],
    )
    def agg_kernel(cur_hbm, packed_hbm, out_hbm,
                   agg_sh, packed_v, sb0, db0, sb1, db1, rows0, rows1,
                   sem0, sem1):
        c = lax.axis_index("c")
        s = lax.axis_index("s")
        wid = s * NC + c

        @pl.loop(0, EDGE_BLK)
        def _(r):
            @pl.loop(0, DIM, step=16)
            def _(col):
                rows0[r, pl.ds(col, 16)] = jnp.zeros((16,), jnp.float32)

        @pl.loop(0, ROWS_PER_TILE // EDGE_BLK)
        def _(j):
            pltpu.sync_copy(
                rows0,
                agg_sh.at[pl.ds(s * ROWS_PER_TILE + j * EDGE_BLK, EDGE_BLK)])

        pltpu.sync_copy(packed_hbm.at[wid], packed_v)

        plsc.subcore_barrier()

        def unpack(i, sb, db):
            @pl.loop(0, EDGE_BLK, step=16)
            def _(j):
                v = packed_v[i, pl.ds(j, 16)]
                sb[pl.ds(j, 16)] = jnp.right_shift(v, 16)
                db[pl.ds(j, 16)] = jnp.bitwise_and(v, 0xFFFF)

        def start_gather(sb, rows, sem):
            pltpu.async_copy(cur_hbm.at[sb], rows, sem)

        def wait_gather(rows, sem):
            pltpu.make_async_copy(
                cur_hbm.at[pl.ds(0, EDGE_BLK)], rows, sem).wait()

        def scatter(rows, db):
            pltpu.sync_copy(rows, agg_sh.at[db], add=True)

        unpack(0, sb0, db0)
        start_gather(sb0, rows0, sem0)

        @pl.loop(0, (N_IT - 1) // 2)
        def _(g):
            i = 2 * g
            unpack(i + 1, sb1, db1)
            start_gather(sb1, rows1, sem1)
            wait_gather(rows0, sem0)
            scatter(rows0, db0)
            unpack(i + 2, sb0, db0)
            start_gather(sb0, rows0, sem0)
            wait_gather(rows1, sem1)
            scatter(rows1, db1)

        wait_gather(rows0, sem0)
        scatter(rows0, db0)

        plsc.subcore_barrier()

        pltpu.sync_copy(
            agg_sh.at[pl.ds(s * ROWS_PER_TILE, ROWS_PER_TILE)],
            out_hbm.at[c, pl.ds(s * ROWS_PER_TILE, ROWS_PER_TILE)])

    return agg_kernel(cur, packed3)


def _mlp_layer(cur, parts, w1, b1, w2, b2, scale, relu):
    def body(scale_ref, cur_ref, p_ref, w1_ref, b1_ref, w2_ref, b2_ref,
             out_ref):
        sc = scale_ref[0]
        p = p_ref[...]
        h = cur_ref[...] * sc + p[0] + p[1]
        h = jnp.dot(h, w1_ref[...], preferred_element_type=jnp.float32)
        h = jnp.maximum(h + b1_ref[...], 0.0)
        h = jnp.dot(h, w2_ref[...], preferred_element_type=jnp.float32)
        h = h + b2_ref[...]
        if relu:
            h = jnp.maximum(h, 0.0)
        out_ref[...] = h

    return pl.pallas_call(
        body,
        grid=(GRID,),
        in_specs=[
            pl.BlockSpec(memory_space=pltpu.SMEM),
            pl.BlockSpec((ROW_BLK, DIM), lambda i: (i, 0)),
            pl.BlockSpec((NC, ROW_BLK, DIM), lambda i: (0, i, 0)),
            pl.BlockSpec((DIM, DIM), lambda i: (0, 0)),
            pl.BlockSpec((1, DIM), lambda i: (0, 0)),
            pl.BlockSpec((DIM, DIM), lambda i: (0, 0)),
            pl.BlockSpec((1, DIM), lambda i: (0, 0)),
        ],
        out_specs=pl.BlockSpec((ROW_BLK, DIM), lambda i: (i, 0)),
        out_shape=jax.ShapeDtypeStruct((N_NODES, DIM), jnp.float32),
    )(scale, cur, parts, w1, b1, w2, b2)


def _mlp_final_pool(cur, parts, w1, b1, w2, b2, scale, z1, z2, batch3):
    def body(scale_ref, cur_ref, p_ref, w1_ref, b1_ref, w2_ref, b2_ref,
             b_ref, z1_ref, z2_ref, z_out, g_out, g_acc, c_acc):
        k = pl.program_id(0)
        sc = scale_ref[0]
        p = p_ref[...]
        h = cur_ref[...] * sc + p[0] + p[1]
        h = jnp.dot(h, w1_ref[...], preferred_element_type=jnp.float32)
        h = jnp.maximum(h + b1_ref[...], 0.0)
        h = jnp.dot(h, w2_ref[...], preferred_element_type=jnp.float32)
        h = h + b2_ref[...]
        zb = (z1_ref[...] + z2_ref[...] + h) * (1.0 / 3.0)
        z_out[...] = zb
        bb = b_ref[0]
        oh = (lax.broadcasted_iota(jnp.int32, (N_GRAPHS, ROW_BLK), 0)
              == jnp.broadcast_to(bb, (N_GRAPHS, ROW_BLK))).astype(jnp.float32)

        @pl.when(k == 0)
        def _():
            g_acc[...] = jnp.zeros_like(g_acc)
            c_acc[...] = jnp.zeros_like(c_acc)

        g_acc[...] += jnp.dot(oh, zb, preferred_element_type=jnp.float32)
        c_acc[...] += jnp.broadcast_to(
            jnp.sum(oh, axis=1, keepdims=True), (N_GRAPHS, DIM))
        g_out[...] = g_acc[...] / jnp.maximum(c_acc[...], 1.0)

    return pl.pallas_call(
        body,
        grid=(GRID,),
        in_specs=[
            pl.BlockSpec(memory_space=pltpu.SMEM),
            pl.BlockSpec((ROW_BLK, DIM), lambda i: (i, 0)),
            pl.BlockSpec((NC, ROW_BLK, DIM), lambda i: (0, i, 0)),
            pl.BlockSpec((DIM, DIM), lambda i: (0, 0)),
            pl.BlockSpec((1, DIM), lambda i: (0, 0)),
            pl.BlockSpec((DIM, DIM), lambda i: (0, 0)),
            pl.BlockSpec((1, DIM), lambda i: (0, 0)),
            pl.BlockSpec((1, 1, ROW_BLK), lambda i: (i, 0, 0)),
            pl.BlockSpec((ROW_BLK, DIM), lambda i: (i, 0)),
            pl.BlockSpec((ROW_BLK, DIM), lambda i: (i, 0)),
        ],
        out_specs=[
            pl.BlockSpec((ROW_BLK, DIM), lambda i: (i, 0)),
            pl.BlockSpec((N_GRAPHS, DIM), lambda i: (0, 0)),
        ],
        out_shape=[
            jax.ShapeDtypeStruct((N_NODES, DIM), jnp.float32),
            jax.ShapeDtypeStruct((N_GRAPHS, DIM), jnp.float32),
        ],
        scratch_shapes=[
            pltpu.VMEM((N_GRAPHS, DIM), jnp.float32),
            pltpu.VMEM((N_GRAPHS, DIM), jnp.float32),
        ],
    )(scale, cur, parts, w1, b1, w2, b2, batch3, z1, z2)


def kernel(x, edge_index, batch, W1, b1, W2, b2, eps):
    packed3 = ((edge_index[0] << 16) | edge_index[1]).reshape(
        NW, N_IT, EDGE_BLK)
    batch3 = batch.reshape(GRID, 1, ROW_BLK)
    scales = (1.0 + eps).astype(jnp.float32)

    cur = x
    outs = []
    for i in range(N_LAYERS - 1):
        parts = _sc_aggregate(cur, packed3)
        cur = _mlp_layer(cur, parts, W1[i], b1[i].reshape(1, DIM),
                         W2[i], b2[i].reshape(1, DIM),
                         scales[i].reshape(1), relu=True)
        outs.append(cur)

    i = N_LAYERS - 1
    parts = _sc_aggregate(cur, packed3)
    z, g = _mlp_final_pool(cur, parts, W1[i], b1[i].reshape(1, DIM),
                           W2[i], b2[i].reshape(1, DIM),
                           scales[i].reshape(1), outs[0], outs[1], batch3)
    return (z, g)


# speedup vs baseline: 1.8510x; 1.0079x over previous
import functools

import jax
import jax.numpy as jnp
from jax import lax
from jax.experimental import pallas as pl
from jax.experimental.pallas import tpu as pltpu
from jax.experimental.pallas import tpu_sc as plsc

N_NODES = 10000
N_EDGES = 320000
DIM = 128
N_LAYERS = 3
N_GRAPHS = 64

NC = 2
NS = 16
NW = NC * NS
EDGES_PER_TILE = N_EDGES // NW
EDGE_BLK = 80
N_IT = EDGES_PER_TILE // EDGE_BLK
N_PAD = 10240
ROWS_PER_TILE = N_PAD // NS
ZERO_ROWS = 16

ROW_BLK = 1000
GRID = N_NODES // ROW_BLK


def _sc_aggregate(cur, packed3):
    mesh = plsc.VectorSubcoreMesh(core_axis_name="c", subcore_axis_name="s")

    @functools.partial(
        pl.kernel,
        out_type=jax.ShapeDtypeStruct((NC, N_PAD, DIM), jnp.float32),
        mesh=mesh,
        scratch_types=[
            pltpu.VMEM_SHARED((N_PAD, DIM), jnp.float32),
            pltpu.VMEM((N_IT, EDGE_BLK), jnp.int32),
            pltpu.VMEM((EDGE_BLK,), jnp.int32),
            pltpu.VMEM((EDGE_BLK,), jnp.int32),
            pltpu.VMEM((EDGE_BLK,), jnp.int32),
            pltpu.VMEM((EDGE_BLK,), jnp.int32),
            pltpu.VMEM((EDGE_BLK, DIM), jnp.float32),
            pltpu.VMEM((EDGE_BLK, DIM), jnp.float32),
            pltpu.SemaphoreType.DMA,
            pltpu.SemaphoreType.DMA,
        ],
    )
    def agg_kernel(cur_hbm, packed_hbm, out_hbm,
                   agg_sh, packed_v, sb0, db0, sb1, db1, rows0, rows1,
                   sem0, sem1):
        c = lax.axis_index("c")
        s = lax.axis_index("s")
        wid = s * NC + c

        def unpack(i, sb, db):
            @pl.loop(0, EDGE_BLK, step=16)
            def _(j):
                v = packed_v[i, pl.ds(j, 16)]
                sb[pl.ds(j, 16)] = jnp.right_shift(v, 16)
                db[pl.ds(j, 16)] = jnp.bitwise_and(v, 0xFFFF)

        def start_gather(sb, rows, sem):
            pltpu.async_copy(cur_hbm.at[sb], rows, sem)

        def wait_gather(rows, sem):
            pltpu.make_async_copy(
                cur_hbm.at[pl.ds(0, EDGE_BLK)], rows, sem).wait()

        def scatter(rows, db):
            pltpu.sync_copy(rows, agg_sh.at[db], add=True)

        pltpu.sync_copy(packed_hbm.at[wid], packed_v)
        unpack(0, sb0, db0)
        start_gather(sb0, rows0, sem0)

        @pl.loop(0, EDGE_BLK)
        def _(r):
            @pl.loop(0, DIM, step=16)
            def _(col):
                rows1[r, pl.ds(col, 16)] = jnp.zeros((16,), jnp.float32)

        @pl.loop(0, ROWS_PER_TILE // EDGE_BLK)
        def _(j):
            pltpu.sync_copy(
                rows1,
                agg_sh.at[pl.ds(s * ROWS_PER_TILE + j * EDGE_BLK, EDGE_BLK)])

        plsc.subcore_barrier()

        @pl.loop(0, (N_IT - 1) // 2)
        def _(g):
            i = 2 * g
            unpack(i + 1, sb1, db1)
            start_gather(sb1, rows1, sem1)
            wait_gather(rows0, sem0)
            scatter(rows0, db0)
            unpack(i + 2, sb0, db0)
            start_gather(sb0, rows0, sem0)
            wait_gather(rows1, sem1)
            scatter(rows1, db1)

        wait_gather(rows0, sem0)
        scatter(rows0, db0)

        plsc.subcore_barrier()

        pltpu.sync_copy(
            agg_sh.at[pl.ds(s * ROWS_PER_TILE, ROWS_PER_TILE)],
            out_hbm.at[c, pl.ds(s * ROWS_PER_TILE, ROWS_PER_TILE)])

    return agg_kernel(cur, packed3)


def _mlp_layer(cur, parts, w1, b1, w2, b2, scale, relu):
    def body(scale_ref, cur_ref, p_ref, w1_ref, b1_ref, w2_ref, b2_ref,
             out_ref):
        sc = scale_ref[0]
        p = p_ref[...]
        h = cur_ref[...] * sc + p[0] + p[1]
        h = jnp.dot(h, w1_ref[...], preferred_element_type=jnp.float32)
        h = jnp.maximum(h + b1_ref[...], 0.0)
        h = jnp.dot(h, w2_ref[...], preferred_element_type=jnp.float32)
        h = h + b2_ref[...]
        if relu:
            h = jnp.maximum(h, 0.0)
        out_ref[...] = h

    return pl.pallas_call(
        body,
        grid=(GRID,),
        in_specs=[
            pl.BlockSpec(memory_space=pltpu.SMEM),
            pl.BlockSpec((ROW_BLK, DIM), lambda i: (i, 0)),
            pl.BlockSpec((NC, ROW_BLK, DIM), lambda i: (0, i, 0)),
            pl.BlockSpec((DIM, DIM), lambda i: (0, 0)),
            pl.BlockSpec((1, DIM), lambda i: (0, 0)),
            pl.BlockSpec((DIM, DIM), lambda i: (0, 0)),
            pl.BlockSpec((1, DIM), lambda i: (0, 0)),
        ],
        out_specs=pl.BlockSpec((ROW_BLK, DIM), lambda i: (i, 0)),
        out_shape=jax.ShapeDtypeStruct((N_NODES, DIM), jnp.float32),
    )(scale, cur, parts, w1, b1, w2, b2)


def _mlp_final_pool(cur, parts, w1, b1, w2, b2, scale, z1, z2, batch3):
    def body(scale_ref, cur_ref, p_ref, w1_ref, b1_ref, w2_ref, b2_ref,
             b_ref, z1_ref, z2_ref, z_out, g_out, g_acc, c_acc):
        k = pl.program_id(0)
        sc = scale_ref[0]
        p = p_ref[...]
        h = cur_ref[...] * sc + p[0] + p[1]
        h = jnp.dot(h, w1_ref[...], preferred_element_type=jnp.float32)
        h = jnp.maximum(h + b1_ref[...], 0.0)
        h = jnp.dot(h, w2_ref[...], preferred_element_type=jnp.float32)
        h = h + b2_ref[...]
        zb = (z1_ref[...] + z2_ref[...] + h) * (1.0 / 3.0)
        z_out[...] = zb
        bb = b_ref[0]
        oh = (lax.broadcasted_iota(jnp.int32, (N_GRAPHS, ROW_BLK), 0)
              == jnp.broadcast_to(bb, (N_GRAPHS, ROW_BLK))).astype(jnp.float32)

        @pl.when(k == 0)
        def _():
            g_acc[...] = jnp.zeros_like(g_acc)
            c_acc[...] = jnp.zeros_like(c_acc)

        g_acc[...] += jnp.dot(oh, zb, preferred_element_type=jnp.float32)
        c_acc[...] += jnp.broadcast_to(
            jnp.sum(oh, axis=1, keepdims=True), (N_GRAPHS, DIM))
        g_out[...] = g_acc[...] / jnp.maximum(c_acc[...], 1.0)

    return pl.pallas_call(
        body,
        grid=(GRID,),
        in_specs=[
            pl.BlockSpec(memory_space=pltpu.SMEM),
            pl.BlockSpec((ROW_BLK, DIM), lambda i: (i, 0)),
            pl.BlockSpec((NC, ROW_BLK, DIM), lambda i: (0, i, 0)),
            pl.BlockSpec((DIM, DIM), lambda i: (0, 0)),
            pl.BlockSpec((1, DIM), lambda i: (0, 0)),
            pl.BlockSpec((DIM, DIM), lambda i: (0, 0)),
            pl.BlockSpec((1, DIM), lambda i: (0, 0)),
            pl.BlockSpec((1, 1, ROW_BLK), lambda i: (i, 0, 0)),
            pl.BlockSpec((ROW_BLK, DIM), lambda i: (i, 0)),
            pl.BlockSpec((ROW_BLK, DIM), lambda i: (i, 0)),
        ],
        out_specs=[
            pl.BlockSpec((ROW_BLK, DIM), lambda i: (i, 0)),
            pl.BlockSpec((N_GRAPHS, DIM), lambda i: (0, 0)),
        ],
        out_shape=[
            jax.ShapeDtypeStruct((N_NODES, DIM), jnp.float32),
            jax.ShapeDtypeStruct((N_GRAPHS, DIM), jnp.float32),
        ],
        scratch_shapes=[
            pltpu.VMEM((N_GRAPHS, DIM), jnp.float32),
            pltpu.VMEM((N_GRAPHS, DIM), jnp.float32),
        ],
    )(scale, cur, parts, w1, b1, w2, b2, batch3, z1, z2)


def kernel(x, edge_index, batch, W1, b1, W2, b2, eps):
    packed3 = ((edge_index[0] << 16) | edge_index[1]).reshape(
        NW, N_IT, EDGE_BLK)
    batch3 = batch.reshape(GRID, 1, ROW_BLK)
    scales = (1.0 + eps).astype(jnp.float32)

    cur = x
    outs = []
    for i in range(N_LAYERS - 1):
        parts = _sc_aggregate(cur, packed3)
        cur = _mlp_layer(cur, parts, W1[i], b1[i].reshape(1, DIM),
                         W2[i], b2[i].reshape(1, DIM),
                         scales[i].reshape(1), relu=True)
        outs.append(cur)

    i = N_LAYERS - 1
    parts = _sc_aggregate(cur, packed3)
    z, g = _mlp_final_pool(cur, parts, W1[i], b1[i].reshape(1, DIM),
                           W2[i], b2[i].reshape(1, DIM),
                           scales[i].reshape(1), outs[0], outs[1], batch3)
    return (z, g)


# speedup vs baseline: 1.8650x; 1.0075x over previous
import functools

import jax
import jax.numpy as jnp
from jax import lax
from jax.experimental import pallas as pl
from jax.experimental.pallas import tpu as pltpu
from jax.experimental.pallas import tpu_sc as plsc

N_NODES = 10000
N_EDGES = 320000
DIM = 128
N_LAYERS = 3
N_GRAPHS = 64

NC = 2
NS = 16
NW = NC * NS
EDGES_PER_TILE = N_EDGES // NW
EDGE_BLK = 80
N_IT = EDGES_PER_TILE // EDGE_BLK
N_PAD = 10240
ROWS_PER_TILE = N_PAD // NS
ZERO_ROWS = 16

ROW_BLK = 1000
GRID = N_NODES // ROW_BLK


def _sc_aggregate(cur, packed3):
    mesh = plsc.VectorSubcoreMesh(core_axis_name="c", subcore_axis_name="s")

    @functools.partial(
        pl.kernel,
        out_type=jax.ShapeDtypeStruct((NC, N_PAD, DIM), jnp.float32),
        mesh=mesh,
        scratch_types=[
            pltpu.VMEM_SHARED((N_PAD, DIM), jnp.float32),
            pltpu.VMEM((N_IT, EDGE_BLK), jnp.int32),
            pltpu.VMEM((4, EDGE_BLK), jnp.int32),
            pltpu.VMEM((4, EDGE_BLK), jnp.int32),
            pltpu.VMEM((EDGE_BLK, DIM), jnp.float32),
            pltpu.VMEM((EDGE_BLK, DIM), jnp.float32),
            pltpu.SemaphoreType.DMA,
            pltpu.SemaphoreType.DMA,
        ],
    )
    def agg_kernel(cur_hbm, packed_hbm, out_hbm,
                   agg_sh, packed_v, sb_v, db_v, rows0, rows1,
                   sem0, sem1):
        c = lax.axis_index("c")
        s = lax.axis_index("s")
        wid = s * NC + c

        def unpack(i, p):
            @pl.loop(0, EDGE_BLK, step=16)
            def _(j):
                v = packed_v[i, pl.ds(j, 16)]
                sb_v[p, pl.ds(j, 16)] = jnp.right_shift(v, 16)
                db_v[p, pl.ds(j, 16)] = jnp.bitwise_and(v, 0xFFFF)

        def start_gather(p, rows, sem):
            pltpu.async_copy(cur_hbm.at[sb_v.at[p]], rows, sem)

        def wait_gather(rows, sem):
            pltpu.make_async_copy(
                cur_hbm.at[pl.ds(0, EDGE_BLK)], rows, sem).wait()

        def scatter(rows, p):
            pltpu.sync_copy(rows, agg_sh.at[db_v.at[p]], add=True)

        pltpu.sync_copy(packed_hbm.at[wid], packed_v)
        unpack(0, 0)
        start_gather(0, rows0, sem0)

        @pl.loop(0, EDGE_BLK)
        def _(r):
            @pl.loop(0, DIM, step=16)
            def _(col):
                rows1[r, pl.ds(col, 16)] = jnp.zeros((16,), jnp.float32)

        @pl.loop(0, ROWS_PER_TILE // EDGE_BLK)
        def _(j):
            pltpu.sync_copy(
                rows1,
                agg_sh.at[pl.ds(s * ROWS_PER_TILE + j * EDGE_BLK, EDGE_BLK)])

        plsc.subcore_barrier()

        unpack(1, 1)
        start_gather(1, rows1, sem1)
        unpack(2, 2)

        @pl.loop(0, (N_IT - 5) // 4)
        def _(g):
            i = 4 * g
            wait_gather(rows0, sem0)
            scatter(rows0, 0)
            start_gather(2, rows0, sem0)
            unpack(i + 3, 3)
            wait_gather(rows1, sem1)
            scatter(rows1, 1)
            start_gather(3, rows1, sem1)
            unpack(i + 4, 0)
            wait_gather(rows0, sem0)
            scatter(rows0, 2)
            start_gather(0, rows0, sem0)
            unpack(i + 5, 1)
            wait_gather(rows1, sem1)
            scatter(rows1, 3)
            start_gather(1, rows1, sem1)
            unpack(i + 6, 2)

        wait_gather(rows0, sem0)
        scatter(rows0, 0)
        start_gather(2, rows0, sem0)
        unpack(N_IT - 2, 3)
        wait_gather(rows1, sem1)
        scatter(rows1, 1)
        start_gather(3, rows1, sem1)
        unpack(N_IT - 1, 0)
        wait_gather(rows0, sem0)
        scatter(rows0, 2)
        start_gather(0, rows0, sem0)
        wait_gather(rows1, sem1)
        scatter(rows1, 3)
        wait_gather(rows0, sem0)
        scatter(rows0, 0)

        plsc.subcore_barrier()

        pltpu.sync_copy(
            agg_sh.at[pl.ds(s * ROWS_PER_TILE, ROWS_PER_TILE)],
            out_hbm.at[c, pl.ds(s * ROWS_PER_TILE, ROWS_PER_TILE)])

    return agg_kernel(cur, packed3)


def _mlp_layer(cur, parts, w1, b1, w2, b2, scale, relu):
    def body(scale_ref, cur_ref, p_ref, w1_ref, b1_ref, w2_ref, b2_ref,
             out_ref):
        sc = scale_ref[0]
        p = p_ref[...]
        h = cur_ref[...] * sc + p[0] + p[1]
        h = jnp.dot(h, w1_ref[...], preferred_element_type=jnp.float32)
        h = jnp.maximum(h + b1_ref[...], 0.0)
        h = jnp.dot(h, w2_ref[...], preferred_element_type=jnp.float32)
        h = h + b2_ref[...]
        if relu:
            h = jnp.maximum(h, 0.0)
        out_ref[...] = h

    return pl.pallas_call(
        body,
        grid=(GRID,),
        in_specs=[
            pl.BlockSpec(memory_space=pltpu.SMEM),
            pl.BlockSpec((ROW_BLK, DIM), lambda i: (i, 0)),
            pl.BlockSpec((NC, ROW_BLK, DIM), lambda i: (0, i, 0)),
            pl.BlockSpec((DIM, DIM), lambda i: (0, 0)),
            pl.BlockSpec((1, DIM), lambda i: (0, 0)),
            pl.BlockSpec((DIM, DIM), lambda i: (0, 0)),
            pl.BlockSpec((1, DIM), lambda i: (0, 0)),
        ],
        out_specs=pl.BlockSpec((ROW_BLK, DIM), lambda i: (i, 0)),
        out_shape=jax.ShapeDtypeStruct((N_NODES, DIM), jnp.float32),
    )(scale, cur, parts, w1, b1, w2, b2)


def _mlp_final_pool(cur, parts, w1, b1, w2, b2, scale, z1, z2, batch3):
    def body(scale_ref, cur_ref, p_ref, w1_ref, b1_ref, w2_ref, b2_ref,
             b_ref, z1_ref, z2_ref, z_out, g_out, g_acc, c_acc):
        k = pl.program_id(0)
        sc = scale_ref[0]
        p = p_ref[...]
        h = cur_ref[...] * sc + p[0] + p[1]
        h = jnp.dot(h, w1_ref[...], preferred_element_type=jnp.float32)
        h = jnp.maximum(h + b1_ref[...], 0.0)
        h = jnp.dot(h, w2_ref[...], preferred_element_type=jnp.float32)
        h = h + b2_ref[...]
        zb = (z1_ref[...] + z2_ref[...] + h) * (1.0 / 3.0)
        z_out[...] = zb
        bb = b_ref[0]
        oh = (lax.broadcasted_iota(jnp.int32, (N_GRAPHS, ROW_BLK), 0)
              == jnp.broadcast_to(bb, (N_GRAPHS, ROW_BLK))).astype(jnp.float32)

        @pl.when(k == 0)
        def _():
            g_acc[...] = jnp.zeros_like(g_acc)
            c_acc[...] = jnp.zeros_like(c_acc)

        g_acc[...] += jnp.dot(oh, zb, preferred_element_type=jnp.float32)
        c_acc[...] += jnp.broadcast_to(
            jnp.sum(oh, axis=1, keepdims=True), (N_GRAPHS, DIM))
        g_out[...] = g_acc[...] / jnp.maximum(c_acc[...], 1.0)

    return pl.pallas_call(
        body,
        grid=(GRID,),
        in_specs=[
            pl.BlockSpec(memory_space=pltpu.SMEM),
            pl.BlockSpec((ROW_BLK, DIM), lambda i: (i, 0)),
            pl.BlockSpec((NC, ROW_BLK, DIM), lambda i: (0, i, 0)),
            pl.BlockSpec((DIM, DIM), lambda i: (0, 0)),
            pl.BlockSpec((1, DIM), lambda i: (0, 0)),
            pl.BlockSpec((DIM, DIM), lambda i: (0, 0)),
            pl.BlockSpec((1, DIM), lambda i: (0, 0)),
            pl.BlockSpec((1, 1, ROW_BLK), lambda i: (i, 0, 0)),
            pl.BlockSpec((ROW_BLK, DIM), lambda i: (i, 0)),
            pl.BlockSpec((ROW_BLK, DIM), lambda i: (i, 0)),
        ],
        out_specs=[
            pl.BlockSpec((ROW_BLK, DIM), lambda i: (i, 0)),
            pl.BlockSpec((N_GRAPHS, DIM), lambda i: (0, 0)),
        ],
        out_shape=[
            jax.ShapeDtypeStruct((N_NODES, DIM), jnp.float32),
            jax.ShapeDtypeStruct((N_GRAPHS, DIM), jnp.float32),
        ],
        scratch_shapes=[
            pltpu.VMEM((N_GRAPHS, DIM), jnp.float32),
            pltpu.VMEM((N_GRAPHS, DIM), jnp.float32),
        ],
    )(scale, cur, parts, w1, b1, w2, b2, batch3, z1, z2)


def kernel(x, edge_index, batch, W1, b1, W2, b2, eps):
    packed3 = ((edge_index[0] << 16) | edge_index[1]).reshape(
        NW, N_IT, EDGE_BLK)
    batch3 = batch.reshape(GRID, 1, ROW_BLK)
    scales = (1.0 + eps).astype(jnp.float32)

    cur = x
    outs = []
    for i in range(N_LAYERS - 1):
        parts = _sc_aggregate(cur, packed3)
        cur = _mlp_layer(cur, parts, W1[i], b1[i].reshape(1, DIM),
                         W2[i], b2[i].reshape(1, DIM),
                         scales[i].reshape(1), relu=True)
        outs.append(cur)

    i = N_LAYERS - 1
    parts = _sc_aggregate(cur, packed3)
    z, g = _mlp_final_pool(cur, parts, W1[i], b1[i].reshape(1, DIM),
                           W2[i], b2[i].reshape(1, DIM),
                           scales[i].reshape(1), outs[0], outs[1], batch3)
    return (z, g)
